# baseline (speedup 1.0000x reference)
import functools

import jax
import jax.numpy as jnp
from jax import lax
from jax.experimental import pallas as pl
from jax.experimental.pallas import tpu as pltpu
from jax.experimental.pallas import tpu_sc as plsc

NUM_POIS = 1000000
EMBED_DIM = 64
BATCH = 16384
HIST = 20

_B = BATCH * HIST
_NW = 32
_G = 128
_NR = _B // _G
_PER_W = _NR // _NW
_HB = BATCH // _G
_TC = 8192
_NBLK = -(-NUM_POIS // (2 * _TC))
_P = _NBLK * _TC


def _transpose_body(a_ref, b_ref, o_ref):
    o_ref[:, 0:EMBED_DIM] = a_ref[...].T
    o_ref[:, EMBED_DIM:128] = b_ref[...].T


def _make_transpose():
    return pl.pallas_call(
        _transpose_body,
        grid=(_NBLK,),
        in_specs=[
            pl.BlockSpec((EMBED_DIM, _TC), lambda g: (0, 2 * g)),
            pl.BlockSpec(
                (EMBED_DIM, _TC),
                lambda g: (0, jnp.minimum(2 * g + 1, 2 * _NBLK - 2))),
        ],
        out_specs=pl.BlockSpec((_TC, 128), lambda g: (g, 0)),
        out_shape=jax.ShapeDtypeStruct((_P, 128), jnp.float32),
    )


def _make_gather():
    mesh = plsc.VectorSubcoreMesh(core_axis_name="c", subcore_axis_name="s")

    @functools.partial(
        pl.kernel,
        mesh=mesh,
        out_type=jax.ShapeDtypeStruct((HIST * 8 * _HB, 8, _G), jnp.float32),
        compiler_params=pltpu.CompilerParams(
            use_tc_tiling_on_sc=False, needs_layout_passes=False),
        scratch_types=(
            [pltpu.VMEM((_PER_W, _G), jnp.int32)]
            + [pltpu.VMEM((_G, EMBED_DIM), jnp.float32)] * 4
            + [pltpu.VMEM((EMBED_DIM, _G), jnp.float32)] * 4
            + [pltpu.SemaphoreType.DMA] * 8
        ),
    )
    def gather_kernel(t2_hbm, idxp_hbm, out_hbm, idxp_v, *bs):
        bufs = bs[0:4]
        slabs = bs[4:8]
        gsem = bs[8:12]
        ssem = bs[12:16]
        wid = lax.axis_index("s") * 2 + lax.axis_index("c")
        r0 = wid * _PER_W
        pltpu.sync_copy(idxp_hbm.at[pl.ds(r0, _PER_W)], idxp_v)

        rows = [lax.iota(jnp.int32, 16) + (16 * g) for g in range(8)]

        def fire_gather(k, b):
            pltpu.async_copy(
                t2_hbm.at[idxp_v.at[k]], bufs[b], gsem[b])

        def wait_gather(k, b):
            pltpu.make_async_copy(
                t2_hbm.at[idxp_v.at[k]], bufs[b], gsem[b]).wait()

        def out_blk(k):
            r = r0 + k
            h = r // _HB
            bt = r - h * _HB
            return h * (8 * _HB) + bt

        def fire_stores(k, b):
            blk = out_blk(k)
            for dt in range(8):
                pltpu.async_copy(
                    slabs[b].at[pl.ds(dt * 8, 8)], out_hbm.at[blk + dt * _HB],
                    ssem[b])

        def wait_stores(k, b):
            blk = out_blk(k)
            for dt in range(8):
                pltpu.make_async_copy(
                    slabs[b].at[pl.ds(dt * 8, 8)], out_hbm.at[blk + dt * _HB],
                    ssem[b]).wait()

        def build_slab(k, b):
            @plsc.parallel_loop(0, EMBED_DIM, unroll=32)
            def dbody(d):
                dvec = rows[0] * 0 + d
                for g in range(8):
                    v = plsc.load_gather(bufs[b], [rows[g], dvec])
                    plsc.store_scatter(slabs[b], [dvec, rows[g]], v)

        for b in range(4):
            fire_gather(b, b)

        @pl.loop(0, _PER_W, step=4)
        def outer(t):
            for b in range(4):
                k = t + b
                wait_gather(k, b)

                @pl.when(k >= 4)
                def _():
                    wait_stores(k - 4, b)

                build_slab(k, b)
                fire_stores(k, b)

                @pl.when(k + 4 < _PER_W)
                def _():
                    fire_gather(k + 4, b)

        for b in range(4):
            wait_stores(_PER_W - 4 + b, b)

    return gather_kernel


_transpose = _make_transpose()
_gather = _make_gather()


def kernel(poi_ids, table):
    table_t = table.T
    t2 = _transpose(table_t, table_t)
    idx = poi_ids.T.reshape(_NR, _G).astype(jnp.int32)
    r = idx % (2 * _TC)
    idxp = (idx // (2 * _TC)) * _TC + (r % _TC)
    offs = (r // _TC) * EMBED_DIM
    t3 = t2.reshape(2 * _P, EMBED_DIM)
    hidx = idxp * 2 + offs // EMBED_DIM
    out3 = _gather(t3, hidx)
    out5 = out3.reshape(HIST, 8, _HB, 8, _G)
    return out5.transpose(2, 4, 0, 1, 3).reshape(BATCH, HIST, EMBED_DIM)


# speedup vs baseline: 1.0123x; 1.0123x over previous
import functools

import jax
import jax.numpy as jnp
from jax import lax
from jax.experimental import pallas as pl
from jax.experimental.pallas import tpu as pltpu
from jax.experimental.pallas import tpu_sc as plsc

NUM_POIS = 1000000
EMBED_DIM = 64
BATCH = 16384
HIST = 20

_B = BATCH * HIST
_NW = 32
_G = 128
_NR = _B // _G
_PER_W = _NR // _NW
_HB = BATCH // _G
_TC = 8192
_NBLK = -(-NUM_POIS // (2 * _TC))
_P = _NBLK * _TC


def _transpose_body(a_ref, b_ref, o_ref):
    o_ref[:, 0:EMBED_DIM] = a_ref[...].T
    o_ref[:, EMBED_DIM:128] = b_ref[...].T


def _make_transpose():
    return pl.pallas_call(
        _transpose_body,
        grid=(_NBLK,),
        in_specs=[
            pl.BlockSpec((EMBED_DIM, _TC), lambda g: (0, 2 * g)),
            pl.BlockSpec(
                (EMBED_DIM, _TC),
                lambda g: (0, jnp.minimum(2 * g + 1, 2 * _NBLK - 2))),
        ],
        out_specs=pl.BlockSpec((_TC, 128), lambda g: (g, 0)),
        out_shape=jax.ShapeDtypeStruct((_P, 128), jnp.float32),
    )


def _make_gather():
    mesh = plsc.VectorSubcoreMesh(core_axis_name="c", subcore_axis_name="s")

    @functools.partial(
        pl.kernel,
        mesh=mesh,
        out_type=jax.ShapeDtypeStruct((HIST * 8 * _HB, 8, _G), jnp.float32),
        compiler_params=pltpu.CompilerParams(
            use_tc_tiling_on_sc=False, needs_layout_passes=False),
        scratch_types=(
            [pltpu.VMEM((_PER_W, _G), jnp.int32)]
            + [pltpu.VMEM((_G, EMBED_DIM), jnp.float32)] * 4
            + [pltpu.VMEM((EMBED_DIM, _G), jnp.float32)] * 4
            + [pltpu.SemaphoreType.DMA] * 8
        ),
    )
    def gather_kernel(t2_hbm, idxp_hbm, out_hbm, idxp_v, *bs):
        bufs = bs[0:4]
        slabs = bs[4:8]
        gsem = bs[8:12]
        ssem = bs[12:16]
        wid = lax.axis_index("s") * 2 + lax.axis_index("c")
        r0 = wid * _PER_W
        pltpu.sync_copy(idxp_hbm.at[pl.ds(r0, _PER_W)], idxp_v)

        rows = [lax.iota(jnp.int32, 16) + (16 * g) for g in range(8)]

        def fire_gather(k, b):
            pltpu.async_copy(
                t2_hbm.at[idxp_v.at[k]], bufs[b], gsem[b])

        def wait_gather(k, b):
            pltpu.make_async_copy(
                t2_hbm.at[idxp_v.at[k]], bufs[b], gsem[b]).wait()

        def out_blk(k):
            r = r0 + k
            h = r // _HB
            bt = r - h * _HB
            return h * (8 * _HB) + bt

        def fire_stores(k, b):
            blk = out_blk(k)
            for dt in range(8):
                pltpu.async_copy(
                    slabs[b].at[pl.ds(dt * 8, 8)], out_hbm.at[blk + dt * _HB],
                    ssem[b])

        def wait_stores(k, b):
            blk = out_blk(k)
            for dt in range(8):
                pltpu.make_async_copy(
                    slabs[b].at[pl.ds(dt * 8, 8)], out_hbm.at[blk + dt * _HB],
                    ssem[b]).wait()

        def build_slab(k, b):
            @plsc.parallel_loop(0, EMBED_DIM, unroll=16)
            def dbody(d):
                dvec = rows[0] * 0 + d
                for g in range(8):
                    v = plsc.load_gather(bufs[b], [rows[g], dvec])
                    plsc.store_scatter(slabs[b], [dvec, rows[g]], v)

        for b in range(4):
            fire_gather(b, b)

        @pl.loop(0, _PER_W, step=4)
        def outer(t):
            for b in range(4):
                k = t + b
                wait_gather(k, b)

                @pl.when(k >= 4)
                def _():
                    wait_stores(k - 4, b)

                build_slab(k, b)
                fire_stores(k, b)

                @pl.when(k + 4 < _PER_W)
                def _():
                    fire_gather(k + 4, b)

        for b in range(4):
            wait_stores(_PER_W - 4 + b, b)

    return gather_kernel


_transpose = _make_transpose()
_gather = _make_gather()


def kernel(poi_ids, table):
    table_t = table.T
    t2 = _transpose(table_t, table_t)
    idx = poi_ids.T.reshape(_NR, _G).astype(jnp.int32)
    r = idx % (2 * _TC)
    idxp = (idx // (2 * _TC)) * _TC + (r % _TC)
    offs = (r // _TC) * EMBED_DIM
    t3 = t2.reshape(2 * _P, EMBED_DIM)
    hidx = idxp * 2 + offs // EMBED_DIM
    out3 = _gather(t3, hidx)
    out5 = out3.reshape(HIST, 8, _HB, 8, _G)
    return out5.transpose(2, 4, 0, 1, 3).reshape(BATCH, HIST, EMBED_DIM)
